# Initial kernel scaffold; baseline (speedup 1.0000x reference)
import jax
import jax.numpy as jnp
from jax.experimental import pallas as pl


def kernel(xyz, points, params):
    raise NotImplementedError("write your pallas kernel here")


# speedup vs baseline: 3.8402x; 3.8402x over previous
import functools

import jax
import jax.numpy as jnp
from jax.experimental import pallas as pl
from jax.experimental.pallas import tpu as pltpu

_NSAMPLE = 32
_BW = 0.1
_EPS = 1e-5

_INTERPRET = False


def _cparams(n_grid):
    return pltpu.CompilerParams(
        dimension_semantics=("arbitrary",) * n_grid,
        vmem_limit_bytes=100 * 1024 * 1024,
    )


def _density_krn(xd_ref, out_ref, *, n, tile):
    t = pl.program_id(1)
    xd = xd_ref[0]
    xs = xd_ref[0, pl.ds(t * tile, tile), :]
    sq_all = jnp.sum(xd * xd, axis=1)[None, :]
    sq_t = jnp.sum(xs * xs, axis=1)[:, None]
    mm = jnp.dot(xs, xd.T, preferred_element_type=jnp.float32)
    d = (-2.0 * mm + sq_t) + sq_all
    g = jnp.exp(-d / (2.0 * _BW * _BW)) / (2.5 * _BW)
    out_ref[0, 0, :] = jnp.sum(g, axis=1) * (1.0 / n)


def _density(xd, n_tiles):
    B, N, _ = xd.shape
    tile = N // n_tiles
    out = pl.pallas_call(
        functools.partial(_density_krn, n=N, tile=tile),
        grid=(B, n_tiles),
        in_specs=[pl.BlockSpec((1, N, 8), lambda b, t: (b, 0, 0))],
        out_specs=pl.BlockSpec((1, 1, tile), lambda b, t: (b, 0, t)),
        out_shape=jax.ShapeDtypeStruct((B, 1, N), jnp.float32),
        compiler_params=_cparams(2),
        interpret=_INTERPRET,
    )(xd)
    return out[:, 0, :]


def _fps_krn(xyz_ref, out_ref, *, npoint, n):
    x = xyz_ref[:, 0, :]
    y = xyz_ref[:, 1, :]
    z = xyz_ref[:, 2, :]
    iota_n = jax.lax.broadcasted_iota(jnp.int32, x.shape, 1).astype(
        jnp.float32)

    def chunk(j, state):
        dist, far = state
        rows = []
        for _ in range(8):
            rows.append(jnp.transpose(far, (1, 0)))
            mask = (iota_n == far).astype(jnp.float32)
            cx = jnp.sum(mask * x, axis=1, keepdims=True)
            cy = jnp.sum(mask * y, axis=1, keepdims=True)
            cz = jnp.sum(mask * z, axis=1, keepdims=True)
            dx = x - cx
            dy = y - cy
            dz = z - cz
            d = dx * dx + dy * dy + dz * dz
            dist = jnp.minimum(dist, d)
            m = jnp.max(dist, axis=1, keepdims=True)
            far = jnp.min(jnp.where(dist == m, iota_n, float(n)), axis=1,
                          keepdims=True)
        out_ref[pl.ds(pl.multiple_of(j * 8, 8), 8), :] = jnp.concatenate(
            rows, axis=0)
        return (dist, far)

    dist0 = iota_n * 0.0 + 1e10
    far0 = jnp.min(iota_n, axis=1, keepdims=True) * 0.0
    jax.lax.fori_loop(0, npoint // 8, chunk, (dist0, far0))


def _fps(xyz, npoint):
    B, _, N = xyz.shape
    out = pl.pallas_call(
        functools.partial(_fps_krn, npoint=npoint, n=N),
        out_shape=jax.ShapeDtypeStruct((npoint, B), jnp.float32),
        compiler_params=_cparams(0),
        interpret=_INTERPRET,
    )(xyz)
    return jnp.transpose(out, (1, 0))


def _dnet_krn(den_ref, w0_ref, b0_ref, g0_ref, be0_ref, w1_ref, b1_ref,
              g1_ref, be1_ref, w2_ref, b2_ref, g2_ref, be2_ref, out_ref):
    x = den_ref[...]

    def bn_relu(y, g, be):
        m = jnp.mean(y)
        yc = y - m
        v = jnp.mean(yc * yc)
        return jnp.maximum(g * yc / jnp.sqrt(v + _EPS) + be, 0.0)

    def b16(v):
        return v.astype(jnp.bfloat16).astype(jnp.float32)

    y1 = [bn_relu(w0_ref[c, 0] * x + b0_ref[c, 0], g0_ref[c, 0],
                  be0_ref[c, 0]) for c in range(8)]
    y1b = [b16(y) for y in y1]
    y2 = []
    for c in range(8):
        acc = b1_ref[c, 0] + jnp.zeros_like(x)
        for i in range(8):
            acc = acc + b16(w1_ref[c, i]) * y1b[i]
        y2.append(bn_relu(acc, g1_ref[c, 0], be1_ref[c, 0]))
    y2b = [b16(y) for y in y2]
    acc3 = b2_ref[0, 0] + jnp.zeros_like(x)
    for i in range(8):
        acc3 = acc3 + b16(w2_ref[0, i]) * y2b[i]
    out_ref[...] = bn_relu(acc3, g2_ref[0, 0], be2_ref[0, 0])


def _dnet(density, p):
    B, N = density.shape
    args = [density]
    for i in range(3):
        args += [p['dn_w%d' % i], p['dn_b%d' % i].reshape(-1, 1),
                 p['dn_g%d' % i].reshape(-1, 1),
                 p['dn_be%d' % i].reshape(-1, 1)]
    return pl.pallas_call(
        _dnet_krn,
        out_shape=jax.ShapeDtypeStruct((B, N), jnp.float32),
        compiler_params=_cparams(0),
        interpret=_INTERPRET,
    )(*args)


def _knn_krn(xd_ref, fps_ref, idx_ref, gxnd_ref, nxd_ref, *, n, stile, k):
    xd = xd_ref[0]
    fps = fps_ref[0]
    iota_sn = jax.lax.broadcasted_iota(jnp.int32, (stile, n), 1).astype(jnp.float32)
    oh = (iota_sn == fps).astype(jnp.float32)
    nxd = jnp.dot(oh, xd, preferred_element_type=jnp.float32,
                precision=jax.lax.Precision.HIGHEST)
    colmask = (jax.lax.broadcasted_iota(jnp.int32, (1, 8), 1) < 3
               ).astype(jnp.float32)
    nx = nxd * colmask
    xm = xd * colmask
    sq_s = jnp.sum(nx * nx, axis=1, keepdims=True)
    sq_n = jnp.sum(xm * xm, axis=1)[None, :]
    mm = jnp.dot(nx, xm.T, preferred_element_type=jnp.float32)
    dist = (-2.0 * mm + sq_s) + sq_n

    def sel_body(kk, dist):
        m = jnp.min(dist, axis=1, keepdims=True)
        sel = jnp.min(jnp.where(dist == m, iota_sn, float(n)), axis=1,
                      keepdims=True)
        idx_ref[0, kk] = sel
        mask = (iota_sn == sel)
        g = jnp.dot(mask.astype(jnp.float32), xd,
                    preferred_element_type=jnp.float32,
                    precision=jax.lax.Precision.HIGHEST)
        gxnd_ref[0, kk] = g - nx
        return jnp.where(mask, 1e30, dist)

    jax.lax.fori_loop(0, k, sel_body, dist)
    nxd_ref[0] = nx


def _knn(xd, fpsf, npoint, k, n_tiles):
    B, N, _ = xd.shape
    st = npoint // n_tiles
    return pl.pallas_call(
        functools.partial(_knn_krn, n=N, stile=st, k=k),
        grid=(B, n_tiles),
        in_specs=[
            pl.BlockSpec((1, N, 8), lambda b, t: (b, 0, 0)),
            pl.BlockSpec((1, st, 1), lambda b, t: (b, t, 0)),
        ],
        out_specs=[
            pl.BlockSpec((1, k, st, 1), lambda b, t: (b, 0, t, 0)),
            pl.BlockSpec((1, k, st, 8), lambda b, t: (b, 0, t, 0)),
            pl.BlockSpec((1, st, 8), lambda b, t: (b, t, 0)),
        ],
        out_shape=[
            jax.ShapeDtypeStruct((B, k, npoint, 1), jnp.float32),
            jax.ShapeDtypeStruct((B, k, npoint, 8), jnp.float32),
            jax.ShapeDtypeStruct((B, npoint, 8), jnp.float32),
        ],
        compiler_params=_cparams(2),
        interpret=_INTERPRET,
    )(xd, fpsf)


def _accum_stats(st_ref, y, first):
    s1 = jnp.sum(y, axis=0, keepdims=True)
    s2 = jnp.sum(y * y, axis=0, keepdims=True)
    part = jnp.concatenate([s1, s2, jnp.zeros((6, y.shape[1]), jnp.float32)],
                           axis=0)

    @pl.when(first)
    def _():
        st_ref[...] = part

    @pl.when(jnp.logical_not(first))
    def _():
        st_ref[...] = st_ref[...] + part


def _bn_apply(y, st, g, be, count):
    m = st[0:1, :] * (1.0 / count)
    v = st[1:2, :] * (1.0 / count) - m * m
    scale = g / jnp.sqrt(v + _EPS)
    return jnp.maximum((y - m) * scale + be, 0.0)


def _gc1_krn(idx_ref, pts_ref, gxn_ref, w1_ref, b1_ref, wn0_ref, wnb0_ref,
             y1_ref, v1_ref, st_ref, wst_ref, *, n, ptile):
    first = (pl.program_id(0) + pl.program_id(1)) == 0
    idxf = idx_ref[0]
    iota = jax.lax.broadcasted_iota(jnp.int32, (ptile, n), 1).astype(jnp.float32)
    oh = (iota == idxf).astype(jnp.float32)
    gp = jnp.dot(oh, pts_ref[0], preferred_element_type=jnp.float32,
                 precision=jax.lax.Precision.HIGHEST)
    gxn = gxn_ref[0]
    xin = jnp.concatenate([gxn[:, :3], gp], axis=1)
    y = jnp.dot(xin, w1_ref[...].T,
                preferred_element_type=jnp.float32) + b1_ref[...]
    y1_ref[0] = y
    _accum_stats(st_ref, y, first)
    v = jnp.dot(gxn, wn0_ref[...].T,
                preferred_element_type=jnp.float32) + wnb0_ref[...]
    v1_ref[0] = v
    _accum_stats(wst_ref, v, first)


def _gconv1(idxf, pts, gxn_flat, w1, b1, wn0, wnb0, n_tiles):
    B, P, _ = idxf.shape
    N, D = pts.shape[1], pts.shape[2]
    C, Ci = w1.shape
    CW = wn0.shape[0]
    pt = P // n_tiles
    return pl.pallas_call(
        functools.partial(_gc1_krn, n=N, ptile=pt),
        grid=(B, n_tiles),
        in_specs=[
            pl.BlockSpec((1, pt, 1), lambda b, t: (b, t, 0)),
            pl.BlockSpec((1, N, D), lambda b, t: (b, 0, 0)),
            pl.BlockSpec((1, pt, 8), lambda b, t: (b, t, 0)),
            pl.BlockSpec((C, Ci), lambda b, t: (0, 0)),
            pl.BlockSpec((1, C), lambda b, t: (0, 0)),
            pl.BlockSpec((CW, 8), lambda b, t: (0, 0)),
            pl.BlockSpec((1, CW), lambda b, t: (0, 0)),
        ],
        out_specs=[
            pl.BlockSpec((1, pt, C), lambda b, t: (b, t, 0)),
            pl.BlockSpec((1, pt, CW), lambda b, t: (b, t, 0)),
            pl.BlockSpec((8, C), lambda b, t: (0, 0)),
            pl.BlockSpec((8, CW), lambda b, t: (0, 0)),
        ],
        out_shape=[
            jax.ShapeDtypeStruct((B, P, C), jnp.float32),
            jax.ShapeDtypeStruct((B, P, CW), jnp.float32),
            jax.ShapeDtypeStruct((8, C), jnp.float32),
            jax.ShapeDtypeStruct((8, CW), jnp.float32),
        ],
        compiler_params=_cparams(2),
        interpret=_INTERPRET,
    )(idxf, pts, gxn_flat, w1, b1, wn0, wnb0)


def _conv_krn(y_ref, st_ref, g_ref, be_ref, w_ref, b_ref,
              v_ref, wst_ref, wg_ref, wbe_ref, ww_ref, wb_ref,
              yo_ref, vo_ref, sto_ref, wsto_ref, *, count):
    first = (pl.program_id(0) + pl.program_id(1)) == 0
    x = _bn_apply(y_ref[0], st_ref[...], g_ref[...], be_ref[...], count)
    y = jnp.dot(x, w_ref[...].T, preferred_element_type=jnp.float32) \
        + b_ref[...]
    yo_ref[0] = y
    _accum_stats(sto_ref, y, first)
    xv = _bn_apply(v_ref[0], wst_ref[...], wg_ref[...], wbe_ref[...], count)
    v = jnp.dot(xv, ww_ref[...].T, preferred_element_type=jnp.float32) \
        + wb_ref[...]
    vo_ref[0] = v
    _accum_stats(wsto_ref, v, first)


def _conv(y, st, g, be, w, b, v, wst, wg, wbe, ww, wb, n_tiles):
    B, P, Ci = y.shape
    Co = w.shape[0]
    CWi = v.shape[2]
    CWo = ww.shape[0]
    pt = P // n_tiles
    count = float(B * P)
    return pl.pallas_call(
        functools.partial(_conv_krn, count=count),
        grid=(B, n_tiles),
        in_specs=[
            pl.BlockSpec((1, pt, Ci), lambda b, t: (b, t, 0)),
            pl.BlockSpec((8, Ci), lambda b, t: (0, 0)),
            pl.BlockSpec((1, Ci), lambda b, t: (0, 0)),
            pl.BlockSpec((1, Ci), lambda b, t: (0, 0)),
            pl.BlockSpec((Co, Ci), lambda b, t: (0, 0)),
            pl.BlockSpec((1, Co), lambda b, t: (0, 0)),
            pl.BlockSpec((1, pt, CWi), lambda b, t: (b, t, 0)),
            pl.BlockSpec((8, CWi), lambda b, t: (0, 0)),
            pl.BlockSpec((1, CWi), lambda b, t: (0, 0)),
            pl.BlockSpec((1, CWi), lambda b, t: (0, 0)),
            pl.BlockSpec((CWo, CWi), lambda b, t: (0, 0)),
            pl.BlockSpec((1, CWo), lambda b, t: (0, 0)),
        ],
        out_specs=[
            pl.BlockSpec((1, pt, Co), lambda b, t: (b, t, 0)),
            pl.BlockSpec((1, pt, CWo), lambda b, t: (b, t, 0)),
            pl.BlockSpec((8, Co), lambda b, t: (0, 0)),
            pl.BlockSpec((8, CWo), lambda b, t: (0, 0)),
        ],
        out_shape=[
            jax.ShapeDtypeStruct((B, P, Co), jnp.float32),
            jax.ShapeDtypeStruct((B, P, CWo), jnp.float32),
            jax.ShapeDtypeStruct((8, Co), jnp.float32),
            jax.ShapeDtypeStruct((8, CWo), jnp.float32),
        ],
        compiler_params=_cparams(2),
        interpret=_INTERPRET,
    )(y, st, g, be, w, b, v, wst, wg, wbe, ww, wb)


def _fin_krn(y_ref, st_ref, g_ref, be_ref, gd_ref, v_ref, wst_ref, wg_ref,
             wbe_ref, lw_ref, lb_ref, z_ref, stz_ref, *, count, stile, k):
    first = (pl.program_id(0) + pl.program_id(1)) == 0
    C = y_ref.shape[2]
    CW = v_ref.shape[2]
    x = _bn_apply(y_ref[0], st_ref[...], g_ref[...], be_ref[...], count)
    x = x * gd_ref[0]
    w = _bn_apply(v_ref[0], wst_ref[...], wg_ref[...], wbe_ref[...], count)
    xs = x.reshape(stile, k, C)
    ws = w.reshape(stile, k, CW)
    out = jax.lax.dot_general(
        xs, ws, dimension_numbers=(((1,), (1,)), ((0,), (0,))),
        preferred_element_type=jnp.float32)
    flat = out.reshape(stile, C * CW)
    z = jnp.dot(flat, lw_ref[...].T,
                preferred_element_type=jnp.float32) + lb_ref[...]
    z_ref[0] = z
    _accum_stats(stz_ref, z, first)


def _final(y3, st3, g3, be3, gd, v3, wst3, wg3, wbe3, lw, lb, npoint, k,
           n_tiles):
    B, P, C = y3.shape
    CW = v3.shape[2]
    Cz = lw.shape[0]
    stile = npoint // n_tiles
    pt = stile * k
    count = float(B * P)
    return pl.pallas_call(
        functools.partial(_fin_krn, count=count, stile=stile, k=k),
        grid=(B, n_tiles),
        in_specs=[
            pl.BlockSpec((1, pt, C), lambda b, t: (b, t, 0)),
            pl.BlockSpec((8, C), lambda b, t: (0, 0)),
            pl.BlockSpec((1, C), lambda b, t: (0, 0)),
            pl.BlockSpec((1, C), lambda b, t: (0, 0)),
            pl.BlockSpec((1, pt, 1), lambda b, t: (b, t, 0)),
            pl.BlockSpec((1, pt, CW), lambda b, t: (b, t, 0)),
            pl.BlockSpec((8, CW), lambda b, t: (0, 0)),
            pl.BlockSpec((1, CW), lambda b, t: (0, 0)),
            pl.BlockSpec((1, CW), lambda b, t: (0, 0)),
            pl.BlockSpec((Cz, C * CW), lambda b, t: (0, 0)),
            pl.BlockSpec((1, Cz), lambda b, t: (0, 0)),
        ],
        out_specs=[
            pl.BlockSpec((1, stile, Cz), lambda b, t: (b, t, 0)),
            pl.BlockSpec((8, Cz), lambda b, t: (0, 0)),
        ],
        out_shape=[
            jax.ShapeDtypeStruct((B, npoint, Cz), jnp.float32),
            jax.ShapeDtypeStruct((8, Cz), jnp.float32),
        ],
        compiler_params=_cparams(2),
        interpret=_INTERPRET,
    )(y3, st3, g3, be3, gd, v3, wst3, wg3, wbe3, lw, lb)


def _bnout_krn(z_ref, st_ref, g_ref, be_ref, out_ref, *, count):
    z = z_ref[...]
    st = st_ref[...]
    m = st[0:1, :] * (1.0 / count)
    v = st[1:2, :] * (1.0 / count) - m * m
    scale = g_ref[...] / jnp.sqrt(v + _EPS)
    out_ref[...] = jnp.maximum((z - m[None]) * scale[None] + be_ref[...][None],
                               0.0)


def _bnout(z, stz, g, be):
    B, S, C = z.shape
    count = float(B * S)
    return pl.pallas_call(
        functools.partial(_bnout_krn, count=count),
        out_shape=jax.ShapeDtypeStruct((B, S, C), jnp.float32),
        compiler_params=_cparams(0),
        interpret=_INTERPRET,
    )(z, stz, g, be)


def kernel(xyz, points, params):
    B, _, N = xyz.shape
    npoint = N // 2
    k = _NSAMPLE
    p = params

    xyz_t = jnp.transpose(xyz, (0, 2, 1))
    pts = jnp.transpose(points, (0, 2, 1))

    xd0 = jnp.pad(xyz_t, ((0, 0), (0, 0), (0, 5)))
    density = _density(xd0, 8)
    fpsf = _fps(xyz, npoint)
    ds = _dnet(density, p)

    xd = jnp.concatenate(
        [xyz_t, ds[:, :, None], jnp.zeros((B, N, 4), jnp.float32)], axis=-1)
    w1 = p['mlp_w0']
    idxf, gxnd, nxd = _knn(xd, fpsf[:, :, None], npoint, k, 4)

    P = npoint * k
    gxn_flat = jnp.transpose(gxnd, (0, 2, 1, 3)).reshape(B, P, 8)
    idx_flat = jnp.transpose(idxf[:, :, :, 0], (0, 2, 1)).reshape(B, P, 1)
    gd = gxn_flat[:, :, 3:4]

    wn0 = jnp.pad(p['wn_w0'], ((0, 0), (0, 5)))
    r1 = lambda a: a.reshape(1, -1)
    y1, v1, st1, wst1 = _gconv1(idx_flat, pts, gxn_flat, w1, r1(p['mlp_b0']),
                                wn0, r1(p['wn_b0']), 16)
    y2, v2, st2, wst2 = _conv(
        y1, st1, r1(p['mlp_g0']), r1(p['mlp_be0']), p['mlp_w1'],
        r1(p['mlp_b1']), v1, wst1, r1(p['wn_g0']), r1(p['wn_be0']),
        p['wn_w1'], r1(p['wn_b1']), 16)
    y3, v3, st3, wst3 = _conv(
        y2, st2, r1(p['mlp_g1']), r1(p['mlp_be1']), p['mlp_w2'],
        r1(p['mlp_b2']), v2, wst2, r1(p['wn_g1']), r1(p['wn_be1']),
        p['wn_w2'], r1(p['wn_b2']), 16)
    z, stz = _final(y3, st3, r1(p['mlp_g2']), r1(p['mlp_be2']), gd,
                    v3, wst3, r1(p['wn_g2']), r1(p['wn_be2']),
                    p['lin_w'], r1(p['lin_b']), npoint, k, 8)
    out = _bnout(z, stz, r1(p['bnl_g']), r1(p['bnl_be']))

    new_xyz_out = jnp.transpose(nxd[:, :, :3], (0, 2, 1))
    return new_xyz_out, jnp.transpose(out, (0, 2, 1))
